# baseline (speedup 1.0000x reference)
import functools

import jax
import jax.numpy as jnp
from jax import lax
from jax.experimental import pallas as pl
from jax.experimental.pallas import tpu as pltpu, tpu_sc as plsc

BATCH = 16384
FEATS = 512
HORIZON = 24

_LOGIT_EPS = 18.420680743952367

_NC = 2
_NS = 16
_NW = _NC * _NS
_BPW = BATCH // _NW
_CHUNK = 128
_NCHUNK = _BPW // _CHUNK


@functools.cache
def _build_sc_gather():
    mesh = plsc.VectorSubcoreMesh(core_axis_name="c", subcore_axis_name="s")

    @functools.partial(
        pl.kernel,
        out_type=jax.ShapeDtypeStruct((2, BATCH), jnp.float32),
        mesh=mesh,
        scratch_types=[
            pltpu.VMEM((_NCHUNK, _CHUNK), jnp.int32),
            pltpu.VMEM((_BPW,), jnp.float32),
            pltpu.VMEM((_BPW,), jnp.float32),
            pltpu.SemaphoreType.DMA,
        ],
    )
    def _sc_gather(ids_hbm, a_hbm, b_hbm, ab_out, idx_v, av, bv, sem):
        wid = lax.axis_index("s") * _NC + lax.axis_index("c")
        base = wid * _BPW
        pltpu.sync_copy(ids_hbm.at[pl.ds(wid * _NCHUNK, _NCHUNK), :], idx_v)
        copies = []
        for j in range(_NCHUNK):
            sl = pl.ds(j * _CHUNK, _CHUNK)
            copies.append(pltpu.async_copy(a_hbm.at[idx_v.at[j]], av.at[sl], sem))
            copies.append(pltpu.async_copy(b_hbm.at[idx_v.at[j]], bv.at[sl], sem))
        for c in copies:
            c.wait()
        pltpu.sync_copy(av, ab_out.at[0, pl.ds(base, _BPW)])
        pltpu.sync_copy(bv, ab_out.at[1, pl.ds(base, _BPW)])

    return _sc_gather


def _tc_head_body(x_ref, wt_ref, bias_ref, zT_ref):
    zT = jax.lax.dot_general(wt_ref[...], x_ref[...], (((1,), (1,)), ((), ())),
                             preferred_element_type=jnp.float32)
    zT_ref[...] = zT + bias_ref[...]


def _tc_cal_body(zT_ref, ab_ref, calT_ref, probsT_ref):
    zT = zT_ref[...]
    probsT_ref[...] = 1.0 / (1.0 + jnp.exp(-zT))
    log_odds = jnp.clip(zT, -_LOGIT_EPS, _LOGIT_EPS)
    a = ab_ref[0:1, :]
    b = ab_ref[1:2, :]
    calT_ref[...] = 1.0 / (1.0 + jnp.exp(-(a * log_odds + b)))


def kernel(past_data, region_ids, W_base, b_base, region_a, region_b):
    flat = past_data.reshape(BATCH, FEATS)
    rid = region_ids.reshape(BATCH // _CHUNK, _CHUNK).astype(jnp.int32)
    ab = _build_sc_gather()(rid, region_a, region_b)

    blk = 4096
    zT = pl.pallas_call(
        _tc_head_body,
        grid=(BATCH // blk,),
        in_specs=[
            pl.BlockSpec((blk, FEATS), lambda i: (i, 0)),
            pl.BlockSpec((HORIZON, FEATS), lambda i: (0, 0)),
            pl.BlockSpec((HORIZON, 1), lambda i: (0, 0)),
        ],
        out_specs=pl.BlockSpec((HORIZON, blk), lambda i: (0, i)),
        out_shape=jax.ShapeDtypeStruct((HORIZON, BATCH), jnp.float32),
        compiler_params=pltpu.CompilerParams(
            dimension_semantics=("parallel",),
        ),
    )(flat, W_base.T, b_base.reshape(HORIZON, 1))

    cblk = 8192
    calT, probsT = pl.pallas_call(
        _tc_cal_body,
        grid=(BATCH // cblk,),
        in_specs=[
            pl.BlockSpec((HORIZON, cblk), lambda i: (0, i)),
            pl.BlockSpec((2, cblk), lambda i: (0, i)),
        ],
        out_specs=[
            pl.BlockSpec((HORIZON, cblk), lambda i: (0, i)),
            pl.BlockSpec((HORIZON, cblk), lambda i: (0, i)),
        ],
        out_shape=[
            jax.ShapeDtypeStruct((HORIZON, BATCH), jnp.float32),
            jax.ShapeDtypeStruct((HORIZON, BATCH), jnp.float32),
        ],
        compiler_params=pltpu.CompilerParams(
            dimension_semantics=("parallel",),
        ),
    )(zT, ab)
    return (calT.T, probsT.T)


# speedup vs baseline: 1.0376x; 1.0376x over previous
import functools

import jax
import jax.numpy as jnp
from jax import lax
from jax.experimental import pallas as pl
from jax.experimental.pallas import tpu as pltpu, tpu_sc as plsc

BATCH = 16384
FEATS = 512
HORIZON = 24

_LOGIT_EPS = 18.420680743952367

_NC = 2
_NS = 16
_NW = _NC * _NS
_BPW = BATCH // _NW
_CHUNK = 128
_NCHUNK = _BPW // _CHUNK


@functools.cache
def _build_sc_gather():
    mesh = plsc.VectorSubcoreMesh(core_axis_name="c", subcore_axis_name="s")

    @functools.partial(
        pl.kernel,
        out_type=jax.ShapeDtypeStruct((2, BATCH), jnp.float32),
        mesh=mesh,
        scratch_types=[
            pltpu.VMEM((_NCHUNK, _CHUNK), jnp.int32),
            pltpu.VMEM((_BPW,), jnp.float32),
            pltpu.VMEM((_BPW,), jnp.float32),
            pltpu.SemaphoreType.DMA,
        ],
    )
    def _sc_gather(ids_hbm, a_hbm, b_hbm, ab_out, idx_v, av, bv, sem):
        wid = lax.axis_index("s") * _NC + lax.axis_index("c")
        base = wid * _BPW
        pltpu.sync_copy(ids_hbm.at[pl.ds(wid * _NCHUNK, _NCHUNK), :], idx_v)
        copies = []
        for j in range(_NCHUNK):
            sl = pl.ds(j * _CHUNK, _CHUNK)
            copies.append(pltpu.async_copy(a_hbm.at[idx_v.at[j]], av.at[sl], sem))
            copies.append(pltpu.async_copy(b_hbm.at[idx_v.at[j]], bv.at[sl], sem))
        for c in copies:
            c.wait()
        pltpu.sync_copy(av, ab_out.at[0, pl.ds(base, _BPW)])
        pltpu.sync_copy(bv, ab_out.at[1, pl.ds(base, _BPW)])

    return _sc_gather


def _tc_body(x_ref, wt_ref, bias_ref, ab_ref, calT_ref, probsT_ref):
    zT = jax.lax.dot_general(wt_ref[...], x_ref[...], (((1,), (1,)), ((), ())),
                             preferred_element_type=jnp.float32)
    zT = zT + bias_ref[...]
    probsT_ref[...] = 1.0 / (1.0 + jnp.exp(-zT))
    log_odds = jnp.clip(zT, -_LOGIT_EPS, _LOGIT_EPS)
    a = ab_ref[0:1, :]
    b = ab_ref[1:2, :]
    calT_ref[...] = 1.0 / (1.0 + jnp.exp(-(a * log_odds + b)))


def kernel(past_data, region_ids, W_base, b_base, region_a, region_b):
    flat = past_data.reshape(BATCH, FEATS)
    rid = region_ids.reshape(BATCH // _CHUNK, _CHUNK).astype(jnp.int32)
    ab = _build_sc_gather()(rid, region_a, region_b)

    blk = 4096
    calT, probsT = pl.pallas_call(
        _tc_body,
        grid=(BATCH // blk,),
        in_specs=[
            pl.BlockSpec((blk, FEATS), lambda i: (i, 0)),
            pl.BlockSpec((HORIZON, FEATS), lambda i: (0, 0)),
            pl.BlockSpec((HORIZON, 1), lambda i: (0, 0)),
            pl.BlockSpec((2, blk), lambda i: (0, i)),
        ],
        out_specs=[
            pl.BlockSpec((HORIZON, blk), lambda i: (0, i)),
            pl.BlockSpec((HORIZON, blk), lambda i: (0, i)),
        ],
        out_shape=[
            jax.ShapeDtypeStruct((HORIZON, BATCH), jnp.float32),
            jax.ShapeDtypeStruct((HORIZON, BATCH), jnp.float32),
        ],
        compiler_params=pltpu.CompilerParams(
            dimension_semantics=("parallel",),
        ),
    )(flat, W_base.T, b_base.reshape(HORIZON, 1), ab)
    return (calT.T, probsT.T)
